# baseline (speedup 1.0000x reference)
import functools

import jax
import jax.numpy as jnp
from jax.experimental import pallas as pl

N = 10000
E = 320000
H = 8


def _mm_block(x_ref, wl_ref, bl_ref, wr_ref, br_ref, xl_ref, xr_ref):
    x = x_ref[...]
    xl_ref[...] = jnp.dot(x, wl_ref[...], preferred_element_type=jnp.float32) + bl_ref[...]
    xr_ref[...] = jnp.dot(x, wr_ref[...], preferred_element_type=jnp.float32) + br_ref[...]


def _dual_matmul(x, Wl, bl, Wr, br):
    n, ic = x.shape
    oc = Wl.shape[1]
    br_rows = 2000
    grid = (n // br_rows,)
    return pl.pallas_call(
        _mm_block,
        grid=grid,
        in_specs=[
            pl.BlockSpec((br_rows, ic), lambda i: (i, 0)),
            pl.BlockSpec((ic, oc), lambda i: (0, 0)),
            pl.BlockSpec((oc,), lambda i: (0,)),
            pl.BlockSpec((ic, oc), lambda i: (0, 0)),
            pl.BlockSpec((oc,), lambda i: (0,)),
        ],
        out_specs=[
            pl.BlockSpec((br_rows, oc), lambda i: (i, 0)),
            pl.BlockSpec((br_rows, oc), lambda i: (i, 0)),
        ],
        out_shape=[
            jax.ShapeDtypeStruct((n, oc), jnp.float32),
            jax.ShapeDtypeStruct((n, oc), jnp.float32),
        ],
    )(x, Wl, bl, Wr, br)


def _gat_layer(x, src, dst, Wl, bl, Wr, br, att, bias):
    C = att.shape[1]
    xl, xr = _dual_matmul(x, Wl, bl, Wr, br)
    xl = xl.reshape(-1, H, C)
    xr = xr.reshape(-1, H, C)
    xj = jnp.take(xl, src, axis=0)
    xi = jnp.take(xr, dst, axis=0)
    e = jax.nn.leaky_relu(xi + xj, negative_slope=0.2)
    logits = jnp.sum(e * att[None, :, :], axis=-1)
    m = jax.ops.segment_max(logits, dst, num_segments=N)
    m = jnp.where(jnp.isfinite(m), m, 0.0)
    ex = jnp.exp(logits - jnp.take(m, dst, axis=0))
    denom = jax.ops.segment_sum(ex, dst, num_segments=N)
    alpha = ex / (jnp.take(denom, dst, axis=0) + 1e-16)
    out = jax.ops.segment_sum(xj * alpha[:, :, None], dst, num_segments=N)
    return out.reshape(N, H * C) + bias


def kernel(x, edgeIds, W1l, b1l, W1r, b1r, att1, bias1, W2l, b2l, W2r, b2r, att2, bias2, W3l, b3l, W3r, b3r, att3, bias3, W4l, b4l, W4r, b4r, att4, bias4):
    xs = jnp.squeeze(x)
    src = edgeIds[0]
    dst = edgeIds[1]
    h = jax.nn.relu(_gat_layer(xs, src, dst, W1l, b1l, W1r, b1r, att1, bias1))
    h = jax.nn.relu(_gat_layer(h, src, dst, W2l, b2l, W2r, b2r, att2, bias2))
    h = jax.nn.relu(_gat_layer(h, src, dst, W3l, b3l, W3r, b3r, att3, bias3))
    h = jax.nn.relu(_gat_layer(h, src, dst, W4l, b4l, W4r, b4r, att4, bias4))
    return jnp.expand_dims(h, axis=0)


# speedup vs baseline: 9.2567x; 9.2567x over previous
import functools

import jax
import jax.numpy as jnp
from jax.experimental import pallas as pl
from jax.experimental.pallas import tpu as pltpu
from jax.experimental.pallas import tpu_sc as plsc

N = 10000
E = 320000
H = 8
D = 256

NC = 2
NS = 16
NT = NC * NS
L = 16

CH = 128
NB = -(-N // NT)
OFFBUF = 344
OFFP_LEN = 10376
NEG = -3.0e38

assert E % CH == 0


def _mm_block(x_ref, wl_ref, bl_ref, wr_ref, br_ref, xl_ref, xr_ref):
    x = x_ref[...]
    xl_ref[...] = jnp.dot(x, wl_ref[...], preferred_element_type=jnp.float32) + bl_ref[...]
    xr_ref[...] = jnp.dot(x, wr_ref[...], preferred_element_type=jnp.float32) + br_ref[...]


def _dual_matmul(x, Wl, bl, Wr, br):
    n, ic = x.shape
    oc = Wl.shape[1]
    br_rows = 2000
    return pl.pallas_call(
        _mm_block,
        grid=(n // br_rows,),
        in_specs=[
            pl.BlockSpec((br_rows, ic), lambda i: (i, 0)),
            pl.BlockSpec((ic, oc), lambda i: (0, 0)),
            pl.BlockSpec((oc,), lambda i: (0,)),
            pl.BlockSpec((ic, oc), lambda i: (0, 0)),
            pl.BlockSpec((oc,), lambda i: (0,)),
        ],
        out_specs=[
            pl.BlockSpec((br_rows, oc), lambda i: (i, 0)),
            pl.BlockSpec((br_rows, oc), lambda i: (i, 0)),
        ],
        out_shape=[
            jax.ShapeDtypeStruct((n, oc), jnp.float32),
            jax.ShapeDtypeStruct((n, oc), jnp.float32),
        ],
    )(x, Wl, bl, Wr, br)


def _gat_sc_kernel():
    mesh = plsc.VectorSubcoreMesh(
        core_axis_name="c", subcore_axis_name="s", num_cores=NC, num_subcores=NS)

    @functools.partial(
        pl.kernel,
        out_type=jax.ShapeDtypeStruct((N, D), jnp.float32),
        mesh=mesh,
        compiler_params=pltpu.CompilerParams(
            use_tc_tiling_on_sc=False, needs_layout_passes=False),
        scratch_types=[
            pltpu.VMEM((CH,), jnp.int32),
            pltpu.VMEM((CH,), jnp.int32),
            pltpu.VMEM((CH, D), jnp.float32),
            pltpu.VMEM((CH, D), jnp.float32),
            pltpu.VMEM((H * CH,), jnp.float32),
            pltpu.VMEM((OFFBUF,), jnp.int32),
            pltpu.VMEM((H * L,), jnp.float32),
            pltpu.VMEM((H * L,), jnp.float32),
            pltpu.VMEM((L * L,), jnp.float32),
            pltpu.VMEM((D,), jnp.float32),
            pltpu.VMEM((D,), jnp.float32),
            pltpu.VMEM((D,), jnp.float32),
            pltpu.VMEM((D,), jnp.float32),
            pltpu.SemaphoreType.DMA,
            pltpu.SemaphoreType.DMA,
        ],
    )
    def gat(xl_h, xr_h, ssrc_h, sdst_h, offp_h, att_h, bias_h, out_h,
            sidx, didx, rxj, rxi, lg, offv, mvec, dvec, wbuf,
            accv, attv, biasv, outrow, sem1, sem2):
        cid = jax.lax.axis_index("c")
        sid = jax.lax.axis_index("s")
        wid = sid * NC + cid
        n0 = wid * NB
        n1 = jnp.minimum(n0 + NB, N)
        a0 = (n0 // 8) * 8
        skew = n0 - a0

        pltpu.sync_copy(offp_h.at[pl.ds(a0, OFFBUF)], offv)
        pltpu.sync_copy(att_h, attv)
        pltpu.sync_copy(bias_h, biasv)

        iot = jax.lax.iota(jnp.int32, L)
        zeros16 = jnp.zeros((L,), jnp.float32)
        neg16 = jnp.full((L,), NEG, jnp.float32)

        def _sload(ref, i):
            return ref[pl.ds(i, L)][0]

        for k8 in range(H):
            mvec[pl.ds(k8 * L, L)] = neg16
            dvec[pl.ds(k8 * L, L)] = zeros16
        for k16 in range(D // L):
            accv[pl.ds(k16 * L, L)] = zeros16

        e0 = _sload(offv, skew)
        e1 = _sload(offv, skew + (n1 - n0))
        kc0 = e0 // CH
        kc1 = (e1 + CH - 1) // CH

        def chunk_body(ki, n_cur):
            base = ki * CH
            pltpu.sync_copy(ssrc_h.at[pl.ds(base, CH)], sidx)
            pltpu.sync_copy(sdst_h.at[pl.ds(base, CH)], didx)
            cp1 = pltpu.async_copy(xl_h.at[sidx], rxj, sem1)
            cp2 = pltpu.async_copy(xr_h.at[didx], rxi, sem2)
            cp1.wait()
            cp2.wait()

            def gh_body(gh, carry):
                g = gh // H
                hh = gh % H
                lanes = g * L + iot
                aw0 = attv[pl.ds(hh * (D // H), L)]
                aw1 = attv[pl.ds(hh * (D // H) + L, L)]
                lacc = zeros16
                for c2 in range(D // H):
                    c = hh * (D // H) + c2
                    cvec = jnp.full_like(iot, c)
                    xjc = plsc.load_gather(rxj, [lanes, cvec])
                    xic = plsc.load_gather(rxi, [lanes, cvec])
                    s = xjc + xic
                    lr = jnp.maximum(s, jnp.float32(0.2) * s)
                    a_c = aw0[c2] if c2 < L else aw1[c2 - L]
                    lacc = lacc + a_c * lr
                plsc.store_scatter(lg, [hh * CH + lanes], lacc)
                return carry
            jax.lax.fori_loop(0, (CH // L) * H, gh_body, 0)

            e_lo = jnp.maximum(e0, base)
            e_hi = jnp.minimum(e1, base + CH)

            def wcond(carry):
                e, n = carry
                return e < e_hi

            def wbody(carry):
                e, n = carry
                seg_start = _sload(offv, skew + (n - n0))
                seg_end = _sload(offv, skew + (n - n0) + 1)
                pe = jnp.minimum(seg_end, e_hi)

                @pl.when(e == seg_start)
                def _():
                    for k8 in range(H):
                        mvec[pl.ds(k8 * L, L)] = neg16
                        dvec[pl.ds(k8 * L, L)] = zeros16
                    for k16 in range(D // L):
                        accv[pl.ds(k16 * L, L)] = zeros16

                ng = (pe - e + (L - 1)) // L

                def g_body(g2, carry2):
                    gbase = e + g2 * L
                    lanes_g = gbase + iot
                    mask = lanes_g < pe
                    loc = jnp.minimum(lanes_g - base, CH - 1)

                    def h2(hh, c3):
                        lgv = plsc.load_gather(lg, [hh * CH + loc])
                        lgv = jnp.where(mask, lgv, NEG)
                        gmax = jnp.max(lgv)
                        mold = mvec[pl.ds(hh * L, L)]
                        mnew = jnp.maximum(mold, gmax)
                        plsc.store_scatter(mvec, [hh * L + iot], mnew)
                        scale = jnp.exp(mold - mnew)
                        exv = jnp.where(mask, jnp.exp(lgv - mnew), jnp.float32(0.0))
                        dv = dvec[pl.ds(hh * L, L)]
                        plsc.store_scatter(dvec, [hh * L + iot], dv * scale + exv)
                        plsc.store_scatter(wbuf, [iot * L + hh], exv)
                        for t in range(2):
                            s0 = hh * 2 * L + t * L
                            av = accv[pl.ds(s0, L)]
                            plsc.store_scatter(accv, [s0 + iot], av * scale)
                        return c3
                    jax.lax.fori_loop(0, H, h2, 0)

                    cnt = jnp.minimum(pe - gbase, L)

                    def j_body(j, c3):
                        ljv = jnp.full_like(iot, gbase + j - base)
                        wrow = wbuf[pl.ds(j * L, L)]
                        for k16 in range(D // L):
                            w = wrow[k16 // 2]
                            xjv = plsc.load_gather(rxj, [ljv, k16 * L + iot])
                            sl = pl.ds(k16 * L, L)
                            accv[sl] = accv[sl] + w * xjv
                        return c3
                    jax.lax.fori_loop(0, cnt, j_body, 0)
                    return carry2
                jax.lax.fori_loop(0, ng, g_body, 0)

                done = pe == seg_end

                @pl.when(done)
                def _():
                    ones16 = jnp.ones((L,), jnp.float32)
                    for hh in range(H):
                        dv = dvec[pl.ds(hh * L, L)]
                        dsum = jnp.full((L,), jnp.sum(dv)) + jnp.float32(1e-16)
                        inv = ones16 / dsum
                        for t in range(2):
                            sl = pl.ds(hh * 2 * L + t * L, L)
                            outrow[sl] = jnp.maximum(
                                accv[sl] * inv + biasv[sl], jnp.float32(0.0))
                    pltpu.sync_copy(outrow, out_h.at[n])

                n_next = jnp.where(done, n + 1, n)
                return (pe, n_next)

            _, n_fin = jax.lax.while_loop(wcond, wbody, (e_lo, n_cur))
            return n_fin

        n_end = jax.lax.fori_loop(kc0, kc1, chunk_body, n0)

        for k16 in range(D // L):
            sl = pl.ds(k16 * L, L)
            outrow[sl] = jnp.maximum(biasv[sl], jnp.float32(0.0))

        def ep_body(n, carry):
            pltpu.sync_copy(outrow, out_h.at[n])
            return carry
        jax.lax.fori_loop(n_end, n1, ep_body, 0)

    return gat


def kernel(x, edgeIds, W1l, b1l, W1r, b1r, att1, bias1, W2l, b2l, W2r, b2r, att2, bias2, W3l, b3l, W3r, b3r, att3, bias3, W4l, b4l, W4r, b4r, att4, bias4):
    xs = jnp.squeeze(x, axis=0)
    src = edgeIds[0]
    dst = edgeIds[1]
    sdst, ssrc = jax.lax.sort_key_val(dst, src)
    off = jnp.searchsorted(sdst, jnp.arange(N + 1, dtype=jnp.int32)).astype(jnp.int32)
    offp = jnp.concatenate([off, jnp.full((OFFP_LEN - (N + 1),), E, jnp.int32)])

    gat = _gat_sc_kernel()
    h = xs
    for (Wl, bl, Wr, br, att, bias) in (
            (W1l, b1l, W1r, b1r, att1, bias1),
            (W2l, b2l, W2r, b2r, att2, bias2),
            (W3l, b3l, W3r, b3r, att3, bias3),
            (W4l, b4l, W4r, b4r, att4, bias4)):
        xl, xr = _dual_matmul(h, Wl, bl, Wr, br)
        h = gat(xl, xr, ssrc, sdst, offp, att.reshape(-1), bias)
    return jnp.expand_dims(h, axis=0)


# speedup vs baseline: 16.9347x; 1.8295x over previous
import functools

import jax
import jax.numpy as jnp
from jax.experimental import pallas as pl
from jax.experimental.pallas import tpu as pltpu
from jax.experimental.pallas import tpu_sc as plsc

N = 10000
E = 320000
H = 8
D = 256

NC = 2
NS = 16
NT = NC * NS
L = 16

CH = 128
NB = -(-N // NT)
OFFBUF = 344
OFFP_LEN = 10376
NEG = -3.0e38

assert E % CH == 0


def _mm_block(x_ref, wl_ref, bl_ref, wr_ref, br_ref, xl_ref, xr_ref):
    x = x_ref[...]
    xl_ref[...] = jnp.dot(x, wl_ref[...], preferred_element_type=jnp.float32) + bl_ref[...]
    xr_ref[...] = jnp.dot(x, wr_ref[...], preferred_element_type=jnp.float32) + br_ref[...]


def _dual_matmul(x, Wl, bl, Wr, br):
    n, ic = x.shape
    oc = Wl.shape[1]
    br_rows = 2000
    return pl.pallas_call(
        _mm_block,
        grid=(n // br_rows,),
        in_specs=[
            pl.BlockSpec((br_rows, ic), lambda i: (i, 0)),
            pl.BlockSpec((ic, oc), lambda i: (0, 0)),
            pl.BlockSpec((oc,), lambda i: (0,)),
            pl.BlockSpec((ic, oc), lambda i: (0, 0)),
            pl.BlockSpec((oc,), lambda i: (0,)),
        ],
        out_specs=[
            pl.BlockSpec((br_rows, oc), lambda i: (i, 0)),
            pl.BlockSpec((br_rows, oc), lambda i: (i, 0)),
        ],
        out_shape=[
            jax.ShapeDtypeStruct((n, oc), jnp.float32),
            jax.ShapeDtypeStruct((n, oc), jnp.float32),
        ],
    )(x, Wl, bl, Wr, br)


def _gat_sc_kernel():
    mesh = plsc.VectorSubcoreMesh(
        core_axis_name="c", subcore_axis_name="s", num_cores=NC, num_subcores=NS)

    @functools.partial(
        pl.kernel,
        out_type=jax.ShapeDtypeStruct((N, D), jnp.float32),
        mesh=mesh,
        compiler_params=pltpu.CompilerParams(
            use_tc_tiling_on_sc=False, needs_layout_passes=False),
        scratch_types=[
            pltpu.VMEM((CH,), jnp.int32),
            pltpu.VMEM((CH,), jnp.int32),
            pltpu.VMEM((CH, D), jnp.float32),
            pltpu.VMEM((CH, D), jnp.float32),
            pltpu.VMEM((H * CH,), jnp.float32),
            pltpu.VMEM((OFFBUF,), jnp.int32),
            pltpu.VMEM((H * L,), jnp.float32),
            pltpu.VMEM((H * L,), jnp.float32),
            pltpu.VMEM((L * L,), jnp.float32),
            pltpu.VMEM((D,), jnp.float32),
            pltpu.VMEM((H * (D // H) * L,), jnp.float32),
            pltpu.VMEM((D,), jnp.float32),
            pltpu.VMEM((D,), jnp.float32),
            pltpu.SemaphoreType.DMA,
            pltpu.SemaphoreType.DMA,
        ],
    )
    def gat(xl_h, xr_h, ssrc_h, sdst_h, offp_h, att_h, bias_h, out_h,
            sidx, didx, rxj, rxi, lg, offv, mvec, dvec, wbuf,
            accv, attv, biasv, outrow, sem1, sem2):
        cid = jax.lax.axis_index("c")
        sid = jax.lax.axis_index("s")
        wid = sid * NC + cid
        n0 = wid * NB
        n1 = jnp.minimum(n0 + NB, N)
        a0 = (n0 // 8) * 8
        skew = n0 - a0

        pltpu.sync_copy(offp_h.at[pl.ds(a0, OFFBUF)], offv)
        pltpu.sync_copy(att_h, attv)
        pltpu.sync_copy(bias_h, biasv)

        iot = jax.lax.iota(jnp.int32, L)
        zeros16 = jnp.zeros((L,), jnp.float32)
        neg16 = jnp.full((L,), NEG, jnp.float32)

        def _sload(ref, i):
            return ref[pl.ds(i, L)][0]

        for k8 in range(H):
            mvec[pl.ds(k8 * L, L)] = neg16
            dvec[pl.ds(k8 * L, L)] = zeros16
        for k16 in range(D // L):
            accv[pl.ds(k16 * L, L)] = zeros16

        e0 = _sload(offv, skew)
        e1 = _sload(offv, skew + (n1 - n0))
        kc0 = e0 // CH
        kc1 = (e1 + CH - 1) // CH

        def chunk_body(ki, n_cur):
            base = ki * CH
            pltpu.sync_copy(ssrc_h.at[pl.ds(base, CH)], sidx)
            pltpu.sync_copy(sdst_h.at[pl.ds(base, CH)], didx)
            cp1 = pltpu.async_copy(xl_h.at[sidx], rxj, sem1)
            cp2 = pltpu.async_copy(xr_h.at[didx], rxi, sem2)
            cp1.wait()
            cp2.wait()

            def gh_body(gh, carry):
                g = gh // H
                hh = gh % H
                lanes = g * L + iot
                lacc = zeros16
                lacc2 = zeros16
                hbase = hh * (D // H)
                for c2 in range(D // H):
                    cvec = hbase + ((c2 + iot) & (D // H - 1))
                    xjc = plsc.load_gather(rxj, [lanes, cvec])
                    xic = plsc.load_gather(rxi, [lanes, cvec])
                    s = xjc + xic
                    lr = jnp.maximum(s, jnp.float32(0.2) * s)
                    av = attv[pl.ds(hbase * L + c2 * L, L)]
                    if c2 % 2 == 0:
                        lacc = lacc + av * lr
                    else:
                        lacc2 = lacc2 + av * lr
                plsc.store_scatter(lg, [hh * CH + lanes], lacc + lacc2)
                return carry
            jax.lax.fori_loop(0, (CH // L) * H, gh_body, 0)

            e_lo = jnp.maximum(e0, base)
            e_hi = jnp.minimum(e1, base + CH)

            def wcond(carry):
                e, n = carry
                return e < e_hi

            def wbody(carry):
                e, n = carry
                seg_start = _sload(offv, skew + (n - n0))
                seg_end = _sload(offv, skew + (n - n0) + 1)
                pe = jnp.minimum(seg_end, e_hi)

                @pl.when(e == seg_start)
                def _():
                    for k8 in range(H):
                        mvec[pl.ds(k8 * L, L)] = neg16
                        dvec[pl.ds(k8 * L, L)] = zeros16
                    for k16 in range(D // L):
                        accv[pl.ds(k16 * L, L)] = zeros16

                ng = (pe - e + (L - 1)) // L

                def g_body(g2, carry2):
                    gbase = e + g2 * L
                    lanes_g = gbase + iot
                    mask = lanes_g < pe
                    loc = jnp.minimum(lanes_g - base, CH - 1)

                    def h2(hh, c3):
                        lgv = plsc.load_gather(lg, [hh * CH + loc])
                        lgv = jnp.where(mask, lgv, NEG)
                        gmax = jnp.max(lgv)
                        mold = mvec[pl.ds(hh * L, L)]
                        mnew = jnp.maximum(mold, gmax)
                        plsc.store_scatter(mvec, [hh * L + iot], mnew)
                        scale = jnp.exp(mold - mnew)
                        exv = jnp.where(mask, jnp.exp(lgv - mnew), jnp.float32(0.0))
                        dv = dvec[pl.ds(hh * L, L)]
                        plsc.store_scatter(dvec, [hh * L + iot], dv * scale + exv)
                        plsc.store_scatter(wbuf, [iot * L + hh], exv)
                        for t in range(2):
                            s0 = hh * 2 * L + t * L
                            av = accv[pl.ds(s0, L)]
                            plsc.store_scatter(accv, [s0 + iot], av * scale)
                        return c3
                    jax.lax.fori_loop(0, H, h2, 0)

                    cnt = jnp.minimum(pe - gbase, L)

                    def j_body(j, c3):
                        ljv = jnp.full_like(iot, gbase + j - base)
                        wrow = wbuf[pl.ds(j * L, L)]
                        for k16 in range(D // L):
                            w = wrow[k16 // 2]
                            xjv = plsc.load_gather(rxj, [ljv, k16 * L + iot])
                            sl = pl.ds(k16 * L, L)
                            accv[sl] = accv[sl] + w * xjv
                        return c3
                    jax.lax.fori_loop(0, cnt, j_body, 0)
                    return carry2
                jax.lax.fori_loop(0, ng, g_body, 0)

                done = pe == seg_end

                @pl.when(done)
                def _():
                    ones16 = jnp.ones((L,), jnp.float32)
                    for hh in range(H):
                        dv = dvec[pl.ds(hh * L, L)]
                        dsum = jnp.full((L,), jnp.sum(dv)) + jnp.float32(1e-16)
                        inv = ones16 / dsum
                        for t in range(2):
                            sl = pl.ds(hh * 2 * L + t * L, L)
                            outrow[sl] = jnp.maximum(
                                accv[sl] * inv + biasv[sl], jnp.float32(0.0))
                    pltpu.sync_copy(outrow, out_h.at[n])

                n_next = jnp.where(done, n + 1, n)
                return (pe, n_next)

            _, n_fin = jax.lax.while_loop(wcond, wbody, (e_lo, n_cur))
            return n_fin

        n_end = jax.lax.fori_loop(kc0, kc1, chunk_body, n0)

        for k16 in range(D // L):
            sl = pl.ds(k16 * L, L)
            outrow[sl] = jnp.maximum(biasv[sl], jnp.float32(0.0))

        def ep_body(n, carry):
            pltpu.sync_copy(outrow, out_h.at[n])
            return carry
        jax.lax.fori_loop(n_end, n1, ep_body, 0)

    return gat


def kernel(x, edgeIds, W1l, b1l, W1r, b1r, att1, bias1, W2l, b2l, W2r, b2r, att2, bias2, W3l, b3l, W3r, b3r, att3, bias3, W4l, b4l, W4r, b4r, att4, bias4):
    xs = jnp.squeeze(x, axis=0)
    src = edgeIds[0]
    dst = edgeIds[1]
    sdst, ssrc = jax.lax.sort_key_val(dst, src)
    off = jnp.searchsorted(sdst, jnp.arange(N + 1, dtype=jnp.int32)).astype(jnp.int32)
    offp = jnp.concatenate([off, jnp.full((OFFP_LEN - (N + 1),), E, jnp.int32)])

    gat = _gat_sc_kernel()
    C = D // H
    rot = (jnp.arange(C)[:, None] + jnp.arange(L)[None, :]) & (C - 1)
    h = xs
    for (Wl, bl, Wr, br, att, bias) in (
            (W1l, b1l, W1r, b1r, att1, bias1),
            (W2l, b2l, W2r, b2r, att2, bias2),
            (W3l, b3l, W3r, b3r, att3, bias3),
            (W4l, b4l, W4r, b4r, att4, bias4)):
        xl, xr = _dual_matmul(h, Wl, bl, Wr, br)
        attrot = att[:, rot].reshape(-1)
        h = gat(xl, xr, ssrc, sdst, offp, attrot, bias)
    return jnp.expand_dims(h, axis=0)


# speedup vs baseline: 19.8598x; 1.1727x over previous
import functools

import jax
import jax.numpy as jnp
from jax.experimental import pallas as pl
from jax.experimental.pallas import tpu as pltpu
from jax.experimental.pallas import tpu_sc as plsc

N = 10000
E = 320000
H = 8
D = 256

NC = 2
NS = 16
NT = NC * NS
L = 16

CH = 96
NCHUNK = -(-E // CH)
EPAD = NCHUNK * CH
NB = -(-N // NT)
OFFBUF = 344
OFFP_LEN = 10376
WST = 17
NEG = -3.0e38


def _mm_block(x_ref, wl_ref, bl_ref, wr_ref, br_ref, xl_ref, xr_ref):
    x = x_ref[...]
    xl_ref[...] = jnp.dot(x, wl_ref[...], preferred_element_type=jnp.float32) + bl_ref[...]
    xr_ref[...] = jnp.dot(x, wr_ref[...], preferred_element_type=jnp.float32) + br_ref[...]


def _dual_matmul(x, Wl, bl, Wr, br):
    n, ic = x.shape
    oc = Wl.shape[1]
    br_rows = 2000
    return pl.pallas_call(
        _mm_block,
        grid=(n // br_rows,),
        in_specs=[
            pl.BlockSpec((br_rows, ic), lambda i: (i, 0)),
            pl.BlockSpec((ic, oc), lambda i: (0, 0)),
            pl.BlockSpec((oc,), lambda i: (0,)),
            pl.BlockSpec((ic, oc), lambda i: (0, 0)),
            pl.BlockSpec((oc,), lambda i: (0,)),
        ],
        out_specs=[
            pl.BlockSpec((br_rows, oc), lambda i: (i, 0)),
            pl.BlockSpec((br_rows, oc), lambda i: (i, 0)),
        ],
        out_shape=[
            jax.ShapeDtypeStruct((n, oc), jnp.float32),
            jax.ShapeDtypeStruct((n, oc), jnp.float32),
        ],
    )(x, Wl, bl, Wr, br)


def _gat_sc_kernel():
    mesh = plsc.VectorSubcoreMesh(
        core_axis_name="c", subcore_axis_name="s", num_cores=NC, num_subcores=NS)

    @functools.partial(
        pl.kernel,
        out_type=jax.ShapeDtypeStruct((N, D), jnp.float32),
        mesh=mesh,
        compiler_params=pltpu.CompilerParams(
            use_tc_tiling_on_sc=False, needs_layout_passes=False),
        scratch_types=[
            pltpu.VMEM((2, CH), jnp.int32),
            pltpu.VMEM((2, CH), jnp.int32),
            pltpu.VMEM((2, CH, D), jnp.float32),
            pltpu.VMEM((2, CH, D), jnp.float32),
            pltpu.VMEM((H * CH,), jnp.float32),
            pltpu.VMEM((OFFBUF,), jnp.int32),
            pltpu.VMEM((H * L,), jnp.float32),
            pltpu.VMEM((H * L,), jnp.float32),
            pltpu.VMEM((L * WST,), jnp.float32),
            pltpu.VMEM((D,), jnp.float32),
            pltpu.VMEM((H * (D // H) * L,), jnp.float32),
            pltpu.VMEM((D,), jnp.float32),
            pltpu.VMEM((D,), jnp.float32),
            pltpu.SemaphoreType.DMA,
            pltpu.SemaphoreType.DMA,
            pltpu.SemaphoreType.DMA,
            pltpu.SemaphoreType.DMA,
        ],
    )
    def gat(xl_h, xr_h, ssrc_h, sdst_h, offp_h, att_h, bias_h, out_h,
            sidx2, didx2, rxj2, rxi2, lg, offv, mvec, dvec, wbuf,
            accv, attv, biasv, outrow, semj0, semi0, semj1, semi1):
        cid = jax.lax.axis_index("c")
        sid = jax.lax.axis_index("s")
        wid = sid * NC + cid
        n0 = wid * NB
        n1 = jnp.minimum(n0 + NB, N)
        a0 = (n0 // 8) * 8
        skew = n0 - a0

        pltpu.sync_copy(offp_h.at[pl.ds(a0, OFFBUF)], offv)
        pltpu.sync_copy(att_h, attv)
        pltpu.sync_copy(bias_h, biasv)

        iot = jax.lax.iota(jnp.int32, L)
        zeros16 = jnp.zeros((L,), jnp.float32)
        neg16 = jnp.full((L,), NEG, jnp.float32)

        def _sload(ref, i):
            return ref[pl.ds(i, L)][0]

        for k8 in range(H):
            mvec[pl.ds(k8 * L, L)] = neg16
            dvec[pl.ds(k8 * L, L)] = zeros16
        for k16 in range(D // L):
            accv[pl.ds(k16 * L, L)] = zeros16

        e0 = _sload(offv, skew)
        e1 = _sload(offv, skew + (n1 - n0))
        kc0 = e0 // CH
        kc1 = (e1 + CH - 1) // CH

        def issue(ki, pb):
            base = ki * CH
            @pl.when(pb == 0)
            def _():
                pltpu.sync_copy(ssrc_h.at[pl.ds(base, CH)], sidx2.at[0])
                pltpu.sync_copy(sdst_h.at[pl.ds(base, CH)], didx2.at[0])
                pltpu.async_copy(xl_h.at[sidx2.at[0]], rxj2.at[0], semj0)
                pltpu.async_copy(xr_h.at[didx2.at[0]], rxi2.at[0], semi0)
            @pl.when(pb == 1)
            def _():
                pltpu.sync_copy(ssrc_h.at[pl.ds(base, CH)], sidx2.at[1])
                pltpu.sync_copy(sdst_h.at[pl.ds(base, CH)], didx2.at[1])
                pltpu.async_copy(xl_h.at[sidx2.at[1]], rxj2.at[1], semj1)
                pltpu.async_copy(xr_h.at[didx2.at[1]], rxi2.at[1], semi1)

        def wait_slot(pb):
            @pl.when(pb == 0)
            def _():
                pltpu.make_async_copy(xl_h.at[sidx2.at[0]], rxj2.at[0], semj0).wait()
                pltpu.make_async_copy(xr_h.at[didx2.at[0]], rxi2.at[0], semi0).wait()
            @pl.when(pb == 1)
            def _():
                pltpu.make_async_copy(xl_h.at[sidx2.at[1]], rxj2.at[1], semj1).wait()
                pltpu.make_async_copy(xr_h.at[didx2.at[1]], rxi2.at[1], semi1).wait()

        @pl.when(kc1 > kc0)
        def _():
            issue(kc0, kc0 & 1)

        def chunk_body(ki, n_cur):
            pb = ki & 1
            base = ki * CH

            @pl.when(ki + 1 < kc1)
            def _():
                issue(ki + 1, 1 - pb)

            wait_slot(pb)
            pbv = jnp.full_like(iot, pb)

            def gh_body(gh, carry):
                g = gh // H
                hh = gh % H
                lanes = g * L + iot
                lacc = zeros16
                lacc2 = zeros16
                hbase = hh * (D // H)
                for c2 in range(D // H):
                    cvec = hbase + ((c2 + iot) & (D // H - 1))
                    xjc = plsc.load_gather(rxj2, [pbv, lanes, cvec])
                    xic = plsc.load_gather(rxi2, [pbv, lanes, cvec])
                    s = xjc + xic
                    lr = jnp.maximum(s, jnp.float32(0.2) * s)
                    av = attv[pl.ds(hbase * L + c2 * L, L)]
                    if c2 % 2 == 0:
                        lacc = lacc + av * lr
                    else:
                        lacc2 = lacc2 + av * lr
                lg[pl.ds(hh * CH + g * L, L)] = lacc + lacc2
                return carry
            jax.lax.fori_loop(0, (CH // L) * H, gh_body, 0)

            e_lo = jnp.maximum(e0, base)
            e_hi = jnp.minimum(e1, base + CH)

            def wcond(carry):
                e, n = carry
                return e < e_hi

            def wbody(carry):
                e, n = carry
                seg_start = _sload(offv, skew + (n - n0))
                seg_end = _sload(offv, skew + (n - n0) + 1)
                pe = jnp.minimum(seg_end, e_hi)

                @pl.when(e == seg_start)
                def _():
                    for k8 in range(H):
                        mvec[pl.ds(k8 * L, L)] = neg16
                        dvec[pl.ds(k8 * L, L)] = zeros16
                    for k16 in range(D // L):
                        accv[pl.ds(k16 * L, L)] = zeros16

                ng = (pe - e + (L - 1)) // L

                def g_body(g2, carry2):
                    gbase = e + g2 * L
                    lanes_g = gbase + iot
                    mask = lanes_g < pe
                    loc = jnp.minimum(lanes_g - base, CH - 1)

                    def h2(hh, c3):
                        lgv = plsc.load_gather(lg, [hh * CH + loc])
                        lgv = jnp.where(mask, lgv, NEG)
                        gmax = jnp.max(lgv)
                        mold = mvec[pl.ds(hh * L, L)]
                        mnew = jnp.maximum(mold, gmax)
                        mvec[pl.ds(hh * L, L)] = mnew
                        scale = jnp.exp(mold - mnew)
                        exv = jnp.where(mask, jnp.exp(lgv - mnew), jnp.float32(0.0))
                        dv = dvec[pl.ds(hh * L, L)]
                        dvec[pl.ds(hh * L, L)] = dv * scale + exv
                        plsc.store_scatter(wbuf, [iot * WST + hh], exv)
                        for t in range(2):
                            s0 = hh * 2 * L + t * L
                            av = accv[pl.ds(s0, L)]
                            accv[pl.ds(s0, L)] = av * scale
                        return c3
                    jax.lax.fori_loop(0, H, h2, 0)

                    cnt = jnp.minimum(pe - gbase, L)

                    def j_body(j, c3):
                        ljv = jnp.full_like(iot, gbase + j - base)
                        wrow = wbuf[pl.ds(j * WST, L)]
                        for k16 in range(D // L):
                            w = wrow[k16 // 2]
                            xjv = plsc.load_gather(rxj2, [pbv, ljv, k16 * L + iot])
                            sl = pl.ds(k16 * L, L)
                            accv[sl] = accv[sl] + w * xjv
                        return c3
                    jax.lax.fori_loop(0, cnt, j_body, 0)
                    return carry2
                jax.lax.fori_loop(0, ng, g_body, 0)

                done = pe == seg_end

                @pl.when(done)
                def _():
                    ones16 = jnp.ones((L,), jnp.float32)
                    for hh in range(H):
                        dv = dvec[pl.ds(hh * L, L)]
                        dsum = jnp.full((L,), jnp.sum(dv)) + jnp.float32(1e-16)
                        inv = ones16 / dsum
                        for t in range(2):
                            sl = pl.ds(hh * 2 * L + t * L, L)
                            outrow[sl] = jnp.maximum(
                                accv[sl] * inv + biasv[sl], jnp.float32(0.0))
                    pltpu.sync_copy(outrow, out_h.at[n])

                n_next = jnp.where(done, n + 1, n)
                return (pe, n_next)

            _, n_fin = jax.lax.while_loop(wcond, wbody, (e_lo, n_cur))
            return n_fin

        n_end = jax.lax.fori_loop(kc0, kc1, chunk_body, n0)

        for k16 in range(D // L):
            sl = pl.ds(k16 * L, L)
            outrow[sl] = jnp.maximum(biasv[sl], jnp.float32(0.0))

        def ep_body(n, carry):
            pltpu.sync_copy(outrow, out_h.at[n])
            return carry
        jax.lax.fori_loop(n_end, n1, ep_body, 0)

    return gat


def kernel(x, edgeIds, W1l, b1l, W1r, b1r, att1, bias1, W2l, b2l, W2r, b2r, att2, bias2, W3l, b3l, W3r, b3r, att3, bias3, W4l, b4l, W4r, b4r, att4, bias4):
    xs = jnp.squeeze(x, axis=0)
    src = edgeIds[0]
    dst = edgeIds[1]
    sdst, ssrc = jax.lax.sort_key_val(dst, src)
    off = jnp.searchsorted(sdst, jnp.arange(N + 1, dtype=jnp.int32)).astype(jnp.int32)
    offp = jnp.concatenate([off, jnp.full((OFFP_LEN - (N + 1),), E, jnp.int32)])
    pad = jnp.zeros((EPAD - E,), jnp.int32)
    ssrc = jnp.concatenate([ssrc, pad])
    sdst = jnp.concatenate([sdst, pad])

    gat = _gat_sc_kernel()
    C = D // H
    rot = (jnp.arange(C)[:, None] + jnp.arange(L)[None, :]) & (C - 1)
    h = xs
    for (Wl, bl, Wr, br, att, bias) in (
            (W1l, b1l, W1r, b1r, att1, bias1),
            (W2l, b2l, W2r, b2r, att2, bias2),
            (W3l, b3l, W3r, b3r, att3, bias3),
            (W4l, b4l, W4r, b4r, att4, bias4)):
        xl, xr = _dual_matmul(h, Wl, bl, Wr, br)
        attrot = att[:, rot].reshape(-1)
        h = gat(xl, xr, ssrc, sdst, offp, attrot, bias)
    return jnp.expand_dims(h, axis=0)


# speedup vs baseline: 20.2295x; 1.0186x over previous
import functools

import jax
import jax.numpy as jnp
from jax.experimental import pallas as pl
from jax.experimental.pallas import tpu as pltpu
from jax.experimental.pallas import tpu_sc as plsc

N = 10000
E = 320000
H = 8
D = 256

NC = 2
NS = 16
NT = NC * NS
L = 16

CH = 96
NCHUNK = -(-E // CH)
EPAD = NCHUNK * CH
NB = -(-N // NT)
OFFBUF = 344
OFFP_LEN = 10376
WST = 17
NEG = -3.0e38


def _mm_block(x_ref, wl_ref, bl_ref, wr_ref, br_ref, xl_ref, xr_ref):
    x = x_ref[...]
    xl_ref[...] = jnp.dot(x, wl_ref[...], preferred_element_type=jnp.float32) + bl_ref[...]
    xr_ref[...] = jnp.dot(x, wr_ref[...], preferred_element_type=jnp.float32) + br_ref[...]


def _dual_matmul(x, Wl, bl, Wr, br):
    n, ic = x.shape
    oc = Wl.shape[1]
    br_rows = 2000
    return pl.pallas_call(
        _mm_block,
        grid=(n // br_rows,),
        in_specs=[
            pl.BlockSpec((br_rows, ic), lambda i: (i, 0)),
            pl.BlockSpec((ic, oc), lambda i: (0, 0)),
            pl.BlockSpec((oc,), lambda i: (0,)),
            pl.BlockSpec((ic, oc), lambda i: (0, 0)),
            pl.BlockSpec((oc,), lambda i: (0,)),
        ],
        out_specs=[
            pl.BlockSpec((br_rows, oc), lambda i: (i, 0)),
            pl.BlockSpec((br_rows, oc), lambda i: (i, 0)),
        ],
        out_shape=[
            jax.ShapeDtypeStruct((n, oc), jnp.float32),
            jax.ShapeDtypeStruct((n, oc), jnp.float32),
        ],
    )(x, Wl, bl, Wr, br)


def _gat_sc_kernel():
    mesh = plsc.VectorSubcoreMesh(
        core_axis_name="c", subcore_axis_name="s", num_cores=NC, num_subcores=NS)

    @functools.partial(
        pl.kernel,
        out_type=jax.ShapeDtypeStruct((N, D), jnp.float32),
        mesh=mesh,
        compiler_params=pltpu.CompilerParams(
            use_tc_tiling_on_sc=False, needs_layout_passes=False),
        scratch_types=[
            pltpu.VMEM((2, CH), jnp.int32),
            pltpu.VMEM((2, CH), jnp.int32),
            pltpu.VMEM((2, CH, D), jnp.float32),
            pltpu.VMEM((2, CH, D), jnp.float32),
            pltpu.VMEM((H * CH,), jnp.float32),
            pltpu.VMEM((OFFBUF,), jnp.int32),
            pltpu.VMEM((H * L,), jnp.float32),
            pltpu.VMEM((H * L,), jnp.float32),
            pltpu.VMEM((L * WST,), jnp.float32),
            pltpu.VMEM((D,), jnp.float32),
            pltpu.VMEM((H * (D // H) * L,), jnp.float32),
            pltpu.VMEM((D,), jnp.float32),
            pltpu.VMEM((D,), jnp.float32),
            pltpu.SemaphoreType.DMA,
            pltpu.SemaphoreType.DMA,
            pltpu.SemaphoreType.DMA,
            pltpu.SemaphoreType.DMA,
        ],
    )
    def gat(xl_h, xr_h, ssrc_h, sdst_h, offp_h, att_h, bias_h, out_h,
            sidx2, didx2, rxj2, rxi2, lg, offv, mvec, dvec, wbuf,
            accv, attv, biasv, outrow, semj0, semi0, semj1, semi1):
        cid = jax.lax.axis_index("c")
        sid = jax.lax.axis_index("s")
        wid = sid * NC + cid
        n0 = wid * NB
        n1 = jnp.minimum(n0 + NB, N)
        a0 = (n0 // 8) * 8
        skew = n0 - a0

        pltpu.sync_copy(offp_h.at[pl.ds(a0, OFFBUF)], offv)
        pltpu.sync_copy(att_h, attv)
        pltpu.sync_copy(bias_h, biasv)

        iot = jax.lax.iota(jnp.int32, L)
        zeros16 = jnp.zeros((L,), jnp.float32)
        neg16 = jnp.full((L,), NEG, jnp.float32)

        def _sload(ref, i):
            return ref[pl.ds(i, L)][0]

        for k8 in range(H):
            mvec[pl.ds(k8 * L, L)] = neg16
            dvec[pl.ds(k8 * L, L)] = zeros16
        for k16 in range(D // L):
            accv[pl.ds(k16 * L, L)] = zeros16

        e0 = _sload(offv, skew)
        e1 = _sload(offv, skew + (n1 - n0))
        kc0 = e0 // CH
        kc1 = (e1 + CH - 1) // CH

        def issue(ki, pb):
            base = ki * CH
            @pl.when(pb == 0)
            def _():
                pltpu.sync_copy(ssrc_h.at[pl.ds(base, CH)], sidx2.at[0])
                pltpu.sync_copy(sdst_h.at[pl.ds(base, CH)], didx2.at[0])
                pltpu.async_copy(xl_h.at[sidx2.at[0]], rxj2.at[0], semj0)
                pltpu.async_copy(xr_h.at[didx2.at[0]], rxi2.at[0], semi0)
            @pl.when(pb == 1)
            def _():
                pltpu.sync_copy(ssrc_h.at[pl.ds(base, CH)], sidx2.at[1])
                pltpu.sync_copy(sdst_h.at[pl.ds(base, CH)], didx2.at[1])
                pltpu.async_copy(xl_h.at[sidx2.at[1]], rxj2.at[1], semj1)
                pltpu.async_copy(xr_h.at[didx2.at[1]], rxi2.at[1], semi1)

        def wait_slot(pb):
            @pl.when(pb == 0)
            def _():
                pltpu.make_async_copy(xl_h.at[sidx2.at[0]], rxj2.at[0], semj0).wait()
                pltpu.make_async_copy(xr_h.at[didx2.at[0]], rxi2.at[0], semi0).wait()
            @pl.when(pb == 1)
            def _():
                pltpu.make_async_copy(xl_h.at[sidx2.at[1]], rxj2.at[1], semj1).wait()
                pltpu.make_async_copy(xr_h.at[didx2.at[1]], rxi2.at[1], semi1).wait()

        @pl.when(kc1 > kc0)
        def _():
            issue(kc0, kc0 & 1)

        def chunk_body(ki, n_cur):
            pb = ki & 1
            base = ki * CH

            @pl.when(ki + 1 < kc1)
            def _():
                issue(ki + 1, 1 - pb)

            wait_slot(pb)
            pbv = jnp.full_like(iot, pb)

            def gh_body(gh, carry):
                g = gh // H
                hh = gh % H
                lanes = g * L + iot
                lacc = zeros16
                lacc2 = zeros16
                hbase = hh * (D // H)
                for c2 in range(D // H):
                    cvec = hbase + ((c2 + iot) & (D // H - 1))
                    xjc = plsc.load_gather(rxj2, [pbv, lanes, cvec])
                    xic = plsc.load_gather(rxi2, [pbv, lanes, cvec])
                    s = xjc + xic
                    lr = jnp.maximum(s, jnp.float32(0.2) * s)
                    av = attv[pl.ds(hbase * L + c2 * L, L)]
                    if c2 % 2 == 0:
                        lacc = lacc + av * lr
                    else:
                        lacc2 = lacc2 + av * lr
                lg[pl.ds(hh * CH + g * L, L)] = lacc + lacc2
                return carry
            jax.lax.fori_loop(0, (CH // L) * H, gh_body, 0)

            e_lo = jnp.maximum(e0, base)
            e_hi = jnp.minimum(e1, base + CH)

            def wcond(carry):
                e, n = carry
                return e < e_hi

            def wbody(carry):
                e, n = carry
                seg_start = _sload(offv, skew + (n - n0))
                seg_end = _sload(offv, skew + (n - n0) + 1)
                pe = jnp.minimum(seg_end, e_hi)

                @pl.when(e == seg_start)
                def _():
                    for k8 in range(H):
                        mvec[pl.ds(k8 * L, L)] = neg16
                        dvec[pl.ds(k8 * L, L)] = zeros16
                    for k16 in range(D // L):
                        accv[pl.ds(k16 * L, L)] = zeros16

                ng = (pe - e + (L - 1)) // L

                def g_body(g2, carry2):
                    gbase = e + g2 * L
                    lanes_g = gbase + iot
                    mask = lanes_g < pe
                    loc = jnp.minimum(lanes_g - base, CH - 1)

                    lgvs = []
                    molds = []
                    mnews = []
                    chg = jnp.float32(-1.0)
                    for hh in range(H):
                        lgv = plsc.load_gather(lg, [hh * CH + loc])
                        lgv = jnp.where(mask, lgv, NEG)
                        gmax = jnp.max(lgv)
                        mold = mvec[pl.ds(hh * L, L)]
                        mnew = jnp.maximum(mold, gmax)
                        mvec[pl.ds(hh * L, L)] = mnew
                        chg = jnp.maximum(chg, jnp.max(mnew - mold))
                        lgvs.append(lgv)
                        molds.append(mold)
                        mnews.append(mnew)

                    @pl.when(chg > 0)
                    def _():
                        for hh in range(H):
                            scale = jnp.exp(molds[hh] - mnews[hh])
                            dvec[pl.ds(hh * L, L)] = dvec[pl.ds(hh * L, L)] * scale
                            for t in range(2):
                                s0 = hh * 2 * L + t * L
                                accv[pl.ds(s0, L)] = accv[pl.ds(s0, L)] * scale

                    for hh in range(H):
                        exv = jnp.where(
                            mask, jnp.exp(lgvs[hh] - mnews[hh]), jnp.float32(0.0))
                        dvec[pl.ds(hh * L, L)] = dvec[pl.ds(hh * L, L)] + exv
                        plsc.store_scatter(wbuf, [iot * WST + hh], exv)

                    cnt = jnp.minimum(pe - gbase, L)

                    def j_body(j, c3):
                        ljv = jnp.full_like(iot, gbase + j - base)
                        wrow = wbuf[pl.ds(j * WST, L)]
                        for k16 in range(D // L):
                            w = wrow[k16 // 2]
                            xjv = plsc.load_gather(rxj2, [pbv, ljv, k16 * L + iot])
                            sl = pl.ds(k16 * L, L)
                            accv[sl] = accv[sl] + w * xjv
                        return c3
                    jax.lax.fori_loop(0, cnt, j_body, 0)
                    return carry2
                jax.lax.fori_loop(0, ng, g_body, 0)

                done = pe == seg_end

                @pl.when(done)
                def _():
                    ones16 = jnp.ones((L,), jnp.float32)
                    for hh in range(H):
                        dv = dvec[pl.ds(hh * L, L)]
                        dsum = jnp.full((L,), jnp.sum(dv)) + jnp.float32(1e-16)
                        inv = ones16 / dsum
                        for t in range(2):
                            sl = pl.ds(hh * 2 * L + t * L, L)
                            outrow[sl] = jnp.maximum(
                                accv[sl] * inv + biasv[sl], jnp.float32(0.0))
                    pltpu.sync_copy(outrow, out_h.at[n])

                n_next = jnp.where(done, n + 1, n)
                return (pe, n_next)

            _, n_fin = jax.lax.while_loop(wcond, wbody, (e_lo, n_cur))
            return n_fin

        n_end = jax.lax.fori_loop(kc0, kc1, chunk_body, n0)

        for k16 in range(D // L):
            sl = pl.ds(k16 * L, L)
            outrow[sl] = jnp.maximum(biasv[sl], jnp.float32(0.0))

        def ep_body(n, carry):
            pltpu.sync_copy(outrow, out_h.at[n])
            return carry
        jax.lax.fori_loop(n_end, n1, ep_body, 0)

    return gat


def kernel(x, edgeIds, W1l, b1l, W1r, b1r, att1, bias1, W2l, b2l, W2r, b2r, att2, bias2, W3l, b3l, W3r, b3r, att3, bias3, W4l, b4l, W4r, b4r, att4, bias4):
    xs = jnp.squeeze(x, axis=0)
    src = edgeIds[0]
    dst = edgeIds[1]
    sdst, ssrc = jax.lax.sort_key_val(dst, src)
    off = jnp.searchsorted(sdst, jnp.arange(N + 1, dtype=jnp.int32)).astype(jnp.int32)
    offp = jnp.concatenate([off, jnp.full((OFFP_LEN - (N + 1),), E, jnp.int32)])
    pad = jnp.zeros((EPAD - E,), jnp.int32)
    ssrc = jnp.concatenate([ssrc, pad])
    sdst = jnp.concatenate([sdst, pad])

    gat = _gat_sc_kernel()
    C = D // H
    rot = (jnp.arange(C)[:, None] + jnp.arange(L)[None, :]) & (C - 1)
    h = xs
    for (Wl, bl, Wr, br, att, bias) in (
            (W1l, b1l, W1r, b1r, att1, bias1),
            (W2l, b2l, W2r, b2r, att2, bias2),
            (W3l, b3l, W3r, b3r, att3, bias3),
            (W4l, b4l, W4r, b4r, att4, bias4)):
        xl, xr = _dual_matmul(h, Wl, bl, Wr, br)
        attrot = att[:, rot].reshape(-1)
        h = gat(xl, xr, ssrc, sdst, offp, attrot, bias)
    return jnp.expand_dims(h, axis=0)
